# baseline (speedup 1.0000x reference)
import functools
import numpy as np

import jax
import jax.numpy as jnp
from jax import lax
from jax.experimental import pallas as pl
from jax.experimental.pallas import tpu as pltpu
from jax.experimental.pallas import tpu_sc as plsc

D_MODEL = 1024
N_HEADS = 16
D_FF = 4096
ENC_IN = 21
C_OUT = 21
SEQ_LEN = 1536
PRED_LEN = 512
BUCKET = 64
N_HASHES = 4
L_TOT = SEQ_LEN + PRED_LEN
DH = D_MODEL // N_HEADS
NBK = L_TOT // BUCKET
NB = N_HASHES * NBK
NS = N_HASHES * L_TOT
NC = NS // BUCKET
BATCH = 2
BH = BATCH * N_HEADS
RB = 512
F32 = jnp.float32
BF16 = jnp.bfloat16


def _bdot(a, b, dims):
    return lax.dot_general(a.astype(BF16), b.astype(BF16), (dims, ((), ())),
                           preferred_element_type=F32)


def _pos_embed(L, d):
    position = jnp.arange(L, dtype=F32)[:, None]
    div = jnp.exp(jnp.arange(0, d, 2, dtype=F32) * (-np.log(10000.0) / d))
    pe = jnp.zeros((L, d), F32)
    pe = pe.at[:, 0::2].set(jnp.sin(position * div))
    pe = pe.at[:, 1::2].set(jnp.cos(position * div))
    return pe


def _embed_kern(xin_ref, w_ref, pe_ref, o_ref):
    o_ref[...] = _bdot(xin_ref[...], w_ref[...], ((1,), (0,))) + pe_ref[...]


def _embed(xin, w_in, pe):
    n = xin.shape[0] // RB
    c3 = xin.shape[1]
    return pl.pallas_call(
        _embed_kern,
        grid=(n,),
        in_specs=[
            pl.BlockSpec((RB, c3), lambda i: (i, 0)),
            pl.BlockSpec((c3, D_MODEL), lambda i: (0, 0)),
            pl.BlockSpec((RB, D_MODEL), lambda i: (i, 0)),
        ],
        out_specs=pl.BlockSpec((RB, D_MODEL), lambda i: (i, 0)),
        out_shape=jax.ShapeDtypeStruct((xin.shape[0], D_MODEL), F32),
    )(xin, w_in, pe)


def _qkv_kern(x_ref, wqk_ref, wv_ref, rm_ref, qkv_ref, bkt_ref):
    x = x_ref[0]
    qk = _bdot(x, wqk_ref[0], ((1,), (1,)))
    v = _bdot(x, wv_ref[0], ((1,), (1,)))
    qkv_ref[0, 0, :, 0:DH] = qk
    qkv_ref[0, 0, :, DH:2 * DH] = v
    rot = _bdot(qk, rm_ref[...], ((1,), (0,)))
    for g in range(N_HASHES):
        r = rot[:, g * (NBK // 2):(g + 1) * (NBK // 2)]
        c = jnp.concatenate([r, -r], axis=1)
        mx = jnp.max(c, axis=1, keepdims=True)
        io = lax.broadcasted_iota(jnp.int32, (L_TOT, NBK), 1)
        idx = jnp.min(jnp.where(c == mx, io, NB), axis=1, keepdims=True)
        bkt_ref[0, 0, g, :, 0:1] = idx + g * NBK


def _qkv_buckets(enc, wqk_h, wv_h, rm):
    return pl.pallas_call(
        _qkv_kern,
        grid=(BATCH, N_HEADS),
        in_specs=[
            pl.BlockSpec((1, L_TOT, D_MODEL), lambda b, h: (b, 0, 0)),
            pl.BlockSpec((1, DH, D_MODEL), lambda b, h: (h, 0, 0)),
            pl.BlockSpec((1, DH, D_MODEL), lambda b, h: (h, 0, 0)),
            pl.BlockSpec((DH, DH), lambda b, h: (0, 0)),
        ],
        out_specs=[
            pl.BlockSpec((1, 1, L_TOT, 2 * DH), lambda b, h: (b, h, 0, 0)),
            pl.BlockSpec((1, 1, N_HASHES, L_TOT, 1), lambda b, h: (b, h, 0, 0, 0)),
        ],
        out_shape=[
            jax.ShapeDtypeStruct((BATCH, N_HEADS, L_TOT, 2 * DH), F32),
            jax.ShapeDtypeStruct((BATCH, N_HEADS, N_HASHES, L_TOT, 1), jnp.int32),
        ],
    )(enc, wqk_h, wv_h, rm)


CH = 256


def _undo_kern(bkt_ref, undo_ref):
    io_l = lax.broadcasted_iota(jnp.int32, (CH, NB), 1)
    ri = lax.broadcasted_iota(jnp.int32, (CH, CH), 0)
    ci = lax.broadcasted_iota(jnp.int32, (CH, CH), 1)
    lmat = (ci < ri).astype(BF16)

    def count_body(c, counts):
        bc = bkt_ref[0, pl.ds(c * CH, CH), :]
        oh = (bc == io_l).astype(F32)
        return counts + jnp.sum(oh, axis=0, keepdims=True)

    counts = lax.fori_loop(0, NS // CH, count_body, jnp.zeros((1, NB), F32))
    s = counts
    for sh in (1, 2, 4, 8, 16, 32, 64):
        s = s + jnp.concatenate([jnp.zeros((1, sh), F32), s[:, :NB - sh]], axis=1)
    cum_excl = s - counts

    def rank_body(c, run):
        bc = bkt_ref[0, pl.ds(c * CH, CH), :]
        ohb = bc == io_l
        oh = ohb.astype(F32)
        rank = lax.dot_general(lmat, oh.astype(BF16), (((1,), (0,)), ((), ())),
                               preferred_element_type=F32)
        base = cum_excl + run
        pos = jnp.sum(jnp.where(ohb, rank + base, 0.0), axis=1, keepdims=True)
        undo_ref[0, pl.ds(c * CH, CH), :] = pos.astype(jnp.int32)
        return run + jnp.sum(oh, axis=0, keepdims=True)

    lax.fori_loop(0, NS // CH, rank_body, jnp.zeros((1, NB), F32))


def _undo_sort(bkt):
    return pl.pallas_call(
        _undo_kern,
        grid=(BH,),
        in_specs=[pl.BlockSpec((1, NS, 1), lambda i: (i, 0, 0))],
        out_specs=pl.BlockSpec((1, NS, 1), lambda i: (i, 0, 0)),
        out_shape=jax.ShapeDtypeStruct((BH, NS, 1), jnp.int32),
    )(bkt)


def _attn_kern(sqkv_ref, stc_ref, str_ref, sop_ref):
    def do_chunk(c, kv, ktp):
        q = sqkv_ref[0, pl.ds(c * BUCKET, BUCKET), 0:DH]
        k = kv[:, 0:DH]
        vv = kv[:, DH:2 * DH]
        nrm = jnp.sqrt(jnp.sum(k * k, axis=1, keepdims=True))
        kn = k / jnp.maximum(nrm, 1e-12)
        dots = _bdot(q, kn, ((1,), (1,))) * (DH ** -0.5)
        qt = stc_ref[0, pl.ds(c * BUCKET, BUCKET), :]
        ktc = str_ref[0, pl.ds(c, 1), :]
        dots = jnp.concatenate(
            [jnp.where(qt == ktp, -5e4, dots[:, 0:BUCKET]),
             jnp.where(qt == ktc, -5e4, dots[:, BUCKET:2 * BUCKET])], axis=1)
        mx = jnp.max(dots, axis=1, keepdims=True)
        ex = jnp.exp(dots - mx)
        lse = mx + jnp.log(jnp.sum(ex, axis=1, keepdims=True))
        probs = jnp.exp(dots - lse)
        bo = _bdot(probs, vv, ((1,), (0,)))
        sop_ref[0, pl.ds(c * BUCKET, BUCKET), 0:DH] = bo
        sop_ref[0, pl.ds(c * BUCKET, BUCKET), DH:DH + 16] = jnp.broadcast_to(
            lse, (BUCKET, 16))

    kv0 = jnp.concatenate([sqkv_ref[0, pl.ds(NS - BUCKET, BUCKET), :],
                           sqkv_ref[0, pl.ds(0, BUCKET), :]], axis=0)
    ktp0 = str_ref[0, pl.ds(NC - 1, 1), :]
    do_chunk(0, kv0, ktp0)

    def body(c, _):
        kv = sqkv_ref[0, pl.ds(c * BUCKET - BUCKET, 2 * BUCKET), :]
        ktp = str_ref[0, pl.ds(c - 1, 1), :]
        do_chunk(c, kv, ktp)
        return 0

    lax.fori_loop(1, NC, body, 0)


def _attention(sqkv, st):
    stc = st.reshape(BH, NS, 1)
    strow = st.reshape(BH, NC, BUCKET)
    return pl.pallas_call(
        _attn_kern,
        grid=(BH,),
        in_specs=[
            pl.BlockSpec((1, NS, 2 * DH), lambda i: (i, 0, 0)),
            pl.BlockSpec((1, NS, 1), lambda i: (i, 0, 0)),
            pl.BlockSpec((1, NC, BUCKET), lambda i: (i, 0, 0)),
        ],
        out_specs=pl.BlockSpec((1, NS, 2 * DH), lambda i: (i, 0, 0)),
        out_shape=jax.ShapeDtypeStruct((BH, NS, 2 * DH), F32),
    )(sqkv, stc, strow)


def _combine_kern(og_ref, out_ref):
    for hh in range(2):
        lgs = [og_ref[0, hh, g, :, DH:DH + 1] for g in range(N_HASHES)]
        mx = lgs[0]
        for g in range(1, N_HASHES):
            mx = jnp.maximum(mx, lgs[g])
        ssum = sum(jnp.exp(lg - mx) for lg in lgs)
        lse = mx + jnp.log(ssum)
        acc = jnp.zeros((L_TOT, DH), F32)
        for g in range(N_HASHES):
            acc = acc + jnp.exp(lgs[g] - lse) * og_ref[0, hh, g, :, 0:DH]
        out_ref[0, :, hh * DH:(hh + 1) * DH] = acc


def _combine(og):
    return pl.pallas_call(
        _combine_kern,
        grid=(BATCH, N_HEADS // 2),
        in_specs=[pl.BlockSpec((1, 2, N_HASHES, L_TOT, 2 * DH),
                               lambda b, h: (b, h, 0, 0, 0))],
        out_specs=pl.BlockSpec((1, L_TOT, 2 * DH), lambda b, h: (b, 0, h)),
        out_shape=jax.ShapeDtypeStruct((BATCH, L_TOT, D_MODEL), F32),
    )(og)


def _ln(x, g, b):
    m = jnp.mean(x, axis=1, keepdims=True)
    v = jnp.mean((x - m) ** 2, axis=1, keepdims=True)
    return (x - m) / jnp.sqrt(v + 1e-5) * g + b


def _post_kern(attn_ref, enc_ref, wo_ref, bo_ref, g1_ref, bn1_ref, xr_ref):
    acc = _bdot(attn_ref[0], wo_ref[...], ((1,), (1,)))
    acc = acc + bo_ref[...] + enc_ref[0]
    xr_ref[0] = _ln(acc, g1_ref[...], bn1_ref[...])


def _post_attn(attn, enc, wo, bo, g1, bn1):
    n = L_TOT // RB
    return pl.pallas_call(
        _post_kern,
        grid=(BATCH, n),
        in_specs=[
            pl.BlockSpec((1, RB, D_MODEL), lambda b, i: (b, i, 0)),
            pl.BlockSpec((1, RB, D_MODEL), lambda b, i: (b, i, 0)),
            pl.BlockSpec((D_MODEL, D_MODEL), lambda b, i: (0, 0)),
            pl.BlockSpec((1, D_MODEL), lambda b, i: (0, 0)),
            pl.BlockSpec((1, D_MODEL), lambda b, i: (0, 0)),
            pl.BlockSpec((1, D_MODEL), lambda b, i: (0, 0)),
        ],
        out_specs=pl.BlockSpec((1, RB, D_MODEL), lambda b, i: (b, i, 0)),
        out_shape=jax.ShapeDtypeStruct((BATCH, L_TOT, D_MODEL), F32),
    )(attn, enc, wo, bo, g1, bn1)


FB = D_FF // 1024


def _ffn_kern(xr_ref, w1_ref, b1_ref, w2_ref, b2_ref, g2_ref, bn2_ref, out_ref):
    j = pl.program_id(2)
    y = _bdot(xr_ref[0], w1_ref[...], ((1,), (1,))) + b1_ref[...]
    y = 0.5 * y * (lax.erf(y * (2 ** -0.5)) + 1.0)
    part = _bdot(y, w2_ref[...], ((1,), (1,)))

    @pl.when(j == 0)
    def _():
        out_ref[0] = part

    @pl.when(j > 0)
    def _():
        out_ref[0] += part

    @pl.when(j == FB - 1)
    def _():
        acc = out_ref[0] + b2_ref[...] + xr_ref[0]
        out_ref[0] = _ln(acc, g2_ref[...], bn2_ref[...])


def _ffn(xr, w1, b1, w2, b2, g2, bn2):
    n = L_TOT // RB
    return pl.pallas_call(
        _ffn_kern,
        grid=(BATCH, n, FB),
        in_specs=[
            pl.BlockSpec((1, RB, D_MODEL), lambda b, i, j: (b, i, 0)),
            pl.BlockSpec((1024, D_MODEL), lambda b, i, j: (j, 0)),
            pl.BlockSpec((1, 1024), lambda b, i, j: (0, j)),
            pl.BlockSpec((D_MODEL, 1024), lambda b, i, j: (0, j)),
            pl.BlockSpec((1, D_MODEL), lambda b, i, j: (0, 0)),
            pl.BlockSpec((1, D_MODEL), lambda b, i, j: (0, 0)),
            pl.BlockSpec((1, D_MODEL), lambda b, i, j: (0, 0)),
        ],
        out_specs=pl.BlockSpec((1, RB, D_MODEL), lambda b, i, j: (b, i, 0)),
        out_shape=jax.ShapeDtypeStruct((BATCH, L_TOT, D_MODEL), F32),
    )(xr, w1, b1, w2, b2, g2, bn2)


def _final_kern(enc_ref, gf_ref, bf_ref, wp_ref, bp_ref, out_ref):
    x = _ln(enc_ref[0], gf_ref[...], bf_ref[...])
    out_ref[0] = _bdot(x, wp_ref[...], ((1,), (1,))) + bp_ref[...]


def _final(enc, gf, bf, wp, bp):
    n = L_TOT // RB
    return pl.pallas_call(
        _final_kern,
        grid=(BATCH, n),
        in_specs=[
            pl.BlockSpec((1, RB, D_MODEL), lambda b, i: (b, i, 0)),
            pl.BlockSpec((1, D_MODEL), lambda b, i: (0, 0)),
            pl.BlockSpec((1, D_MODEL), lambda b, i: (0, 0)),
            pl.BlockSpec((C_OUT, D_MODEL), lambda b, i: (0, 0)),
            pl.BlockSpec((1, C_OUT), lambda b, i: (0, 0)),
        ],
        out_specs=pl.BlockSpec((1, RB, C_OUT), lambda b, i: (b, i, 0)),
        out_shape=jax.ShapeDtypeStruct((BATCH, L_TOT, C_OUT), F32),
    )(enc, gf, bf, wp, bp)


CGN = 128


def _sc_sort_gather(undo, qkvr):
    mesh = plsc.VectorSubcoreMesh(core_axis_name="c", subcore_axis_name="s")

    @functools.partial(
        pl.kernel,
        mesh=mesh,
        out_type=[
            jax.ShapeDtypeStruct((BH, NS), jnp.int32),
            jax.ShapeDtypeStruct((BH, NS, 2 * DH), F32),
        ],
        scratch_types=[
            pltpu.VMEM((NS,), jnp.int32),
            pltpu.VMEM((NS,), jnp.int32),
            pltpu.VMEM((CGN, 2 * DH), F32),
            pltpu.SemaphoreType.DMA,
        ],
        compiler_params=pltpu.CompilerParams(needs_layout_passes=False),
    )
    def k(undo_hbm, qkv_hbm, st_hbm, sqkv_hbm, undo_v, st_v, buf, sem):
        w = lax.axis_index("s") * 2 + lax.axis_index("c")
        pltpu.sync_copy(undo_hbm.at[w], undo_v)

        def scat(i, _):
            idx = undo_v[pl.ds(i * 16, 16)]
            val = (lax.iota(jnp.int32, 16) + i * 16) & (L_TOT - 1)
            plsc.store_scatter(st_v, [idx], val)
            return 0

        lax.fori_loop(0, NS // 16, scat, 0)
        pltpu.sync_copy(st_v, st_hbm.at[w])

        def gath(j, _):
            pltpu.async_copy(
                qkv_hbm.at[w].at[st_v.at[pl.ds(j * CGN, CGN)]], buf, sem).wait()
            pltpu.sync_copy(buf, sqkv_hbm.at[w, pl.ds(j * CGN, CGN)])
            return 0

        lax.fori_loop(0, NS // CGN, gath, 0)

    return k(undo, qkvr)


def _sc_unsort_gather(sop, undo):
    mesh = plsc.VectorSubcoreMesh(core_axis_name="c", subcore_axis_name="s")
    D = 2 * DH

    @functools.partial(
        pl.kernel,
        mesh=mesh,
        out_type=jax.ShapeDtypeStruct((BH, NS, D), F32),
        scratch_types=[
            pltpu.VMEM((NS,), jnp.int32),
            pltpu.VMEM((CGN, D), F32),
            pltpu.SemaphoreType.DMA,
        ],
    )
    def k(sop_hbm, undo_hbm, og_hbm, undo_v, buf, sem):
        w = lax.axis_index("s") * 2 + lax.axis_index("c")
        pltpu.sync_copy(undo_hbm.at[w], undo_v)

        def gath(j, _):
            pltpu.async_copy(
                sop_hbm.at[w].at[undo_v.at[pl.ds(j * CGN, CGN)]], buf, sem).wait()
            pltpu.sync_copy(buf, og_hbm.at[w, pl.ds(j * CGN, CGN)])
            return 0

        lax.fori_loop(0, NS // CGN, gath, 0)

    return k(sop, undo)


def _layer(enc, p, rm):
    wqk_h = p['Wqk'].reshape(N_HEADS, DH, D_MODEL)
    wv_h = p['Wv'].reshape(N_HEADS, DH, D_MODEL)
    qkv, bkt = _qkv_buckets(enc, wqk_h, wv_h, rm)
    qkvr = qkv.reshape(BH, L_TOT, 2 * DH)
    bktr = bkt.reshape(BH, NS, 1)
    undo = _undo_sort(bktr)[:, :, 0]
    st, sqkv = _sc_sort_gather(undo, qkvr)
    sop = _attention(sqkv, st)
    og = _sc_unsort_gather(sop, undo)
    attn = _combine(og.reshape(BATCH, N_HEADS, N_HASHES, L_TOT, 2 * DH))
    xr = _post_attn(attn, enc, p['Wo'], p['bo'].reshape(1, D_MODEL),
                    p['g1'].reshape(1, D_MODEL), p['bn1'].reshape(1, D_MODEL))
    return _ffn(xr, p['W1'], p['b1'].reshape(1, D_FF), p['W2'],
                p['b2'].reshape(1, D_MODEL), p['g2'].reshape(1, D_MODEL),
                p['bn2'].reshape(1, D_MODEL))


def kernel(x_enc, x_mark_enc, x_dec, x_mark_dec, params):
    x = jnp.concatenate([x_enc, x_dec[:, -PRED_LEN:, :]], axis=1)
    xm = jnp.concatenate([x_mark_enc, x_mark_dec[:, -PRED_LEN:, :]], axis=1)
    B, L, _ = x.shape
    W = params['conv_token']
    xin = jnp.concatenate([jnp.roll(x, 1, axis=1), x, jnp.roll(x, -1, axis=1), xm],
                          axis=-1)
    w_in = jnp.concatenate([W[:, :, 0].T, W[:, :, 1].T, W[:, :, 2].T,
                            params['W_timef'].T], axis=0)
    pe = _pos_embed(L, D_MODEL)
    pe2 = jnp.broadcast_to(pe[None], (B, L, D_MODEL)).reshape(B * L, D_MODEL)
    enc = _embed(xin.reshape(B * L, -1), w_in, pe2).reshape(B, L, D_MODEL)

    for li, p in enumerate(params['layers']):
        k = jax.random.fold_in(jax.random.key(42), li)
        rot = jax.random.normal(k, (1, DH, N_HASHES, NBK // 2), F32)
        rm = rot[0].reshape(DH, N_HASHES * (NBK // 2))
        enc = _layer(enc, p, rm)

    dec = _final(enc, params['gf'].reshape(1, D_MODEL),
                 params['bf'].reshape(1, D_MODEL), params['Wp'],
                 params['bp'].reshape(1, C_OUT))
    return dec[:, -PRED_LEN:, :]


# speedup vs baseline: 1.9364x; 1.9364x over previous
import functools
import numpy as np

import jax
import jax.numpy as jnp
from jax import lax
from jax.experimental import pallas as pl
from jax.experimental.pallas import tpu as pltpu
from jax.experimental.pallas import tpu_sc as plsc

D_MODEL = 1024
N_HEADS = 16
D_FF = 4096
ENC_IN = 21
C_OUT = 21
SEQ_LEN = 1536
PRED_LEN = 512
BUCKET = 64
N_HASHES = 4
L_TOT = SEQ_LEN + PRED_LEN
DH = D_MODEL // N_HEADS
NBK = L_TOT // BUCKET
NB = N_HASHES * NBK
NS = N_HASHES * L_TOT
NC = NS // BUCKET
BATCH = 2
BH = BATCH * N_HEADS
RB = 512
F32 = jnp.float32
BF16 = jnp.bfloat16


def _bdot(a, b, dims):
    return lax.dot_general(a.astype(BF16), b.astype(BF16), (dims, ((), ())),
                           preferred_element_type=F32)


def _pos_embed(L, d):
    position = jnp.arange(L, dtype=F32)[:, None]
    div = jnp.exp(jnp.arange(0, d, 2, dtype=F32) * (-np.log(10000.0) / d))
    pe = jnp.zeros((L, d), F32)
    pe = pe.at[:, 0::2].set(jnp.sin(position * div))
    pe = pe.at[:, 1::2].set(jnp.cos(position * div))
    return pe


def _embed_kern(xin_ref, w_ref, pe_ref, o_ref):
    o_ref[...] = _bdot(xin_ref[...], w_ref[...], ((1,), (0,))) + pe_ref[...]


def _embed(xin, w_in, pe):
    n = xin.shape[0] // RB
    c3 = xin.shape[1]
    return pl.pallas_call(
        _embed_kern,
        grid=(n,),
        in_specs=[
            pl.BlockSpec((RB, c3), lambda i: (i, 0)),
            pl.BlockSpec((c3, D_MODEL), lambda i: (0, 0)),
            pl.BlockSpec((RB, D_MODEL), lambda i: (i, 0)),
        ],
        out_specs=pl.BlockSpec((RB, D_MODEL), lambda i: (i, 0)),
        out_shape=jax.ShapeDtypeStruct((xin.shape[0], D_MODEL), F32),
    )(xin, w_in, pe)


def _qkv_kern(x_ref, wqk_ref, wv_ref, rm_ref, qkv_ref, bkt_ref):
    x = x_ref[0]
    qk = _bdot(x, wqk_ref[0], ((1,), (1,)))
    v = _bdot(x, wv_ref[0], ((1,), (1,)))
    qkv_ref[0, 0, :, 0:DH] = qk
    qkv_ref[0, 0, :, DH:2 * DH] = v
    rot = _bdot(qk, rm_ref[...], ((1,), (0,)))
    for g in range(N_HASHES):
        r = rot[:, g * (NBK // 2):(g + 1) * (NBK // 2)]
        c = jnp.concatenate([r, -r], axis=1)
        mx = jnp.max(c, axis=1, keepdims=True)
        io = lax.broadcasted_iota(jnp.int32, (L_TOT, NBK), 1)
        idx = jnp.min(jnp.where(c == mx, io, NB), axis=1, keepdims=True)
        bkt_ref[0, 0, g, :, 0:1] = idx + g * NBK


def _qkv_buckets(enc, wqk_h, wv_h, rm):
    return pl.pallas_call(
        _qkv_kern,
        grid=(BATCH, N_HEADS),
        in_specs=[
            pl.BlockSpec((1, L_TOT, D_MODEL), lambda b, h: (b, 0, 0)),
            pl.BlockSpec((1, DH, D_MODEL), lambda b, h: (h, 0, 0)),
            pl.BlockSpec((1, DH, D_MODEL), lambda b, h: (h, 0, 0)),
            pl.BlockSpec((DH, DH), lambda b, h: (0, 0)),
        ],
        out_specs=[
            pl.BlockSpec((1, 1, L_TOT, 2 * DH), lambda b, h: (b, h, 0, 0)),
            pl.BlockSpec((1, 1, N_HASHES, L_TOT, 1), lambda b, h: (b, h, 0, 0, 0)),
        ],
        out_shape=[
            jax.ShapeDtypeStruct((BATCH, N_HEADS, L_TOT, 2 * DH), F32),
            jax.ShapeDtypeStruct((BATCH, N_HEADS, N_HASHES, L_TOT, 1), jnp.int32),
        ],
    )(enc, wqk_h, wv_h, rm)


CH = 256


def _undo_kern(bkt_ref, undo_ref):
    io_l = lax.broadcasted_iota(jnp.int32, (CH, NB), 1)
    ri = lax.broadcasted_iota(jnp.int32, (CH, CH), 0)
    ci = lax.broadcasted_iota(jnp.int32, (CH, CH), 1)
    lmat = (ci < ri).astype(BF16)

    def count_body(c, counts):
        bc = bkt_ref[0, pl.ds(c * CH, CH), :]
        oh = (bc == io_l).astype(F32)
        return counts + jnp.sum(oh, axis=0, keepdims=True)

    counts = lax.fori_loop(0, NS // CH, count_body, jnp.zeros((1, NB), F32))
    s = counts
    for sh in (1, 2, 4, 8, 16, 32, 64):
        s = s + jnp.concatenate([jnp.zeros((1, sh), F32), s[:, :NB - sh]], axis=1)
    cum_excl = s - counts

    def rank_body(c, run):
        bc = bkt_ref[0, pl.ds(c * CH, CH), :]
        ohb = bc == io_l
        oh = ohb.astype(F32)
        rank = lax.dot_general(lmat, oh.astype(BF16), (((1,), (0,)), ((), ())),
                               preferred_element_type=F32)
        base = cum_excl + run
        pos = jnp.sum(jnp.where(ohb, rank + base, 0.0), axis=1, keepdims=True)
        undo_ref[0, pl.ds(c * CH, CH), :] = pos.astype(jnp.int32)
        return run + jnp.sum(oh, axis=0, keepdims=True)

    lax.fori_loop(0, NS // CH, rank_body, jnp.zeros((1, NB), F32))


def _undo_sort(bkt):
    return pl.pallas_call(
        _undo_kern,
        grid=(BH,),
        in_specs=[pl.BlockSpec((1, NS, 1), lambda i: (i, 0, 0))],
        out_specs=pl.BlockSpec((1, NS, 1), lambda i: (i, 0, 0)),
        out_shape=jax.ShapeDtypeStruct((BH, NS, 1), jnp.int32),
    )(bkt)


SCH = 256
SKV = SCH + BUCKET
NSC = NS // SCH


def _attn_kern(sqkv_ref, stc_ref, str_ref, sop_ref):
    qi = lax.broadcasted_iota(jnp.int32, (SCH, SKV), 0) // BUCKET
    kj = lax.broadcasted_iota(jnp.int32, (SCH, SKV), 1) // BUCKET
    d = qi + 1 - kj
    outside = jnp.logical_or(d < 0, d > 1)

    def do_super(s, kv, kt):
        q = sqkv_ref[0, pl.ds(s * SCH, SCH), 0:DH]
        k = kv[:, 0:DH]
        vv = kv[:, DH:2 * DH]
        nrm = jnp.sqrt(jnp.sum(k * k, axis=1, keepdims=True))
        kn = k / jnp.maximum(nrm, 1e-12)
        dots = _bdot(q, kn, ((1,), (1,))) * (DH ** -0.5)
        qt = stc_ref[0, pl.ds(s * SCH, SCH), :]
        dots = jnp.where(qt == kt, -5e4, dots)
        dots = jnp.where(outside, -1e9, dots)
        mx = jnp.max(dots, axis=1, keepdims=True)
        ex = jnp.exp(dots - mx)
        lse = mx + jnp.log(jnp.sum(ex, axis=1, keepdims=True))
        probs = jnp.exp(dots - lse)
        bo = _bdot(probs, vv, ((1,), (0,)))
        sop_ref[0, pl.ds(s * SCH, SCH), 0:DH] = bo
        sop_ref[0, pl.ds(s * SCH, SCH), DH:DH + 16] = jnp.broadcast_to(
            lse, (SCH, 16))

    nps = SCH // BUCKET
    def kt_row(p):
        return str_ref[0, pl.ds(p, 1), :]

    kv0 = jnp.concatenate([sqkv_ref[0, pl.ds(NS - BUCKET, BUCKET), :],
                           sqkv_ref[0, pl.ds(0, SCH), :]], axis=0)
    kt0 = jnp.concatenate([kt_row(NC - 1)] + [kt_row(p) for p in range(nps)],
                          axis=1)
    do_super(0, kv0, kt0)

    def body(s, _):
        kv = sqkv_ref[0, pl.ds(s * SCH - BUCKET, SKV), :]
        kt = jnp.concatenate([kt_row(s * nps - 1 + p) for p in range(nps + 1)],
                             axis=1)
        do_super(s, kv, kt)
        return 0

    lax.fori_loop(1, NSC, body, 0)


def _attention(sqkv, st):
    stc = st.reshape(BH, NS, 1)
    strow = st.reshape(BH, NC, BUCKET)
    return pl.pallas_call(
        _attn_kern,
        grid=(BH,),
        in_specs=[
            pl.BlockSpec((1, NS, 2 * DH), lambda i: (i, 0, 0)),
            pl.BlockSpec((1, NS, 1), lambda i: (i, 0, 0)),
            pl.BlockSpec((1, NC, BUCKET), lambda i: (i, 0, 0)),
        ],
        out_specs=pl.BlockSpec((1, NS, 2 * DH), lambda i: (i, 0, 0)),
        out_shape=jax.ShapeDtypeStruct((BH, NS, 2 * DH), F32),
    )(sqkv, stc, strow)


def _combine_kern(og_ref, out_ref):
    for hh in range(2):
        lgs = [og_ref[0, hh, g, :, DH:DH + 1] for g in range(N_HASHES)]
        mx = lgs[0]
        for g in range(1, N_HASHES):
            mx = jnp.maximum(mx, lgs[g])
        ssum = sum(jnp.exp(lg - mx) for lg in lgs)
        lse = mx + jnp.log(ssum)
        acc = jnp.zeros((L_TOT, DH), F32)
        for g in range(N_HASHES):
            acc = acc + jnp.exp(lgs[g] - lse) * og_ref[0, hh, g, :, 0:DH]
        out_ref[0, :, hh * DH:(hh + 1) * DH] = acc


def _combine(og):
    return pl.pallas_call(
        _combine_kern,
        grid=(BATCH, N_HEADS // 2),
        in_specs=[pl.BlockSpec((1, 2, N_HASHES, L_TOT, 2 * DH),
                               lambda b, h: (b, h, 0, 0, 0))],
        out_specs=pl.BlockSpec((1, L_TOT, 2 * DH), lambda b, h: (b, 0, h)),
        out_shape=jax.ShapeDtypeStruct((BATCH, L_TOT, D_MODEL), F32),
    )(og)


def _ln(x, g, b):
    m = jnp.mean(x, axis=1, keepdims=True)
    v = jnp.mean((x - m) ** 2, axis=1, keepdims=True)
    return (x - m) / jnp.sqrt(v + 1e-5) * g + b


def _post_kern(attn_ref, enc_ref, wo_ref, bo_ref, g1_ref, bn1_ref, xr_ref):
    acc = _bdot(attn_ref[0], wo_ref[...], ((1,), (1,)))
    acc = acc + bo_ref[...] + enc_ref[0]
    xr_ref[0] = _ln(acc, g1_ref[...], bn1_ref[...])


def _post_attn(attn, enc, wo, bo, g1, bn1):
    n = L_TOT // RB
    return pl.pallas_call(
        _post_kern,
        grid=(BATCH, n),
        in_specs=[
            pl.BlockSpec((1, RB, D_MODEL), lambda b, i: (b, i, 0)),
            pl.BlockSpec((1, RB, D_MODEL), lambda b, i: (b, i, 0)),
            pl.BlockSpec((D_MODEL, D_MODEL), lambda b, i: (0, 0)),
            pl.BlockSpec((1, D_MODEL), lambda b, i: (0, 0)),
            pl.BlockSpec((1, D_MODEL), lambda b, i: (0, 0)),
            pl.BlockSpec((1, D_MODEL), lambda b, i: (0, 0)),
        ],
        out_specs=pl.BlockSpec((1, RB, D_MODEL), lambda b, i: (b, i, 0)),
        out_shape=jax.ShapeDtypeStruct((BATCH, L_TOT, D_MODEL), F32),
    )(attn, enc, wo, bo, g1, bn1)


FB = D_FF // 1024


def _ffn_kern(xr_ref, w1_ref, b1_ref, w2_ref, b2_ref, g2_ref, bn2_ref, out_ref):
    j = pl.program_id(2)
    y = _bdot(xr_ref[0], w1_ref[...], ((1,), (1,))) + b1_ref[...]
    y = 0.5 * y * (lax.erf(y * (2 ** -0.5)) + 1.0)
    part = _bdot(y, w2_ref[...], ((1,), (1,)))

    @pl.when(j == 0)
    def _():
        out_ref[0] = part

    @pl.when(j > 0)
    def _():
        out_ref[0] += part

    @pl.when(j == FB - 1)
    def _():
        acc = out_ref[0] + b2_ref[...] + xr_ref[0]
        out_ref[0] = _ln(acc, g2_ref[...], bn2_ref[...])


def _ffn(xr, w1, b1, w2, b2, g2, bn2):
    n = L_TOT // RB
    return pl.pallas_call(
        _ffn_kern,
        grid=(BATCH, n, FB),
        in_specs=[
            pl.BlockSpec((1, RB, D_MODEL), lambda b, i, j: (b, i, 0)),
            pl.BlockSpec((1024, D_MODEL), lambda b, i, j: (j, 0)),
            pl.BlockSpec((1, 1024), lambda b, i, j: (0, j)),
            pl.BlockSpec((D_MODEL, 1024), lambda b, i, j: (0, j)),
            pl.BlockSpec((1, D_MODEL), lambda b, i, j: (0, 0)),
            pl.BlockSpec((1, D_MODEL), lambda b, i, j: (0, 0)),
            pl.BlockSpec((1, D_MODEL), lambda b, i, j: (0, 0)),
        ],
        out_specs=pl.BlockSpec((1, RB, D_MODEL), lambda b, i, j: (b, i, 0)),
        out_shape=jax.ShapeDtypeStruct((BATCH, L_TOT, D_MODEL), F32),
    )(xr, w1, b1, w2, b2, g2, bn2)


def _final_kern(enc_ref, gf_ref, bf_ref, wp_ref, bp_ref, out_ref):
    x = _ln(enc_ref[0], gf_ref[...], bf_ref[...])
    out_ref[0] = _bdot(x, wp_ref[...], ((1,), (1,))) + bp_ref[...]


def _final(enc, gf, bf, wp, bp):
    n = L_TOT // RB
    return pl.pallas_call(
        _final_kern,
        grid=(BATCH, n),
        in_specs=[
            pl.BlockSpec((1, RB, D_MODEL), lambda b, i: (b, i, 0)),
            pl.BlockSpec((1, D_MODEL), lambda b, i: (0, 0)),
            pl.BlockSpec((1, D_MODEL), lambda b, i: (0, 0)),
            pl.BlockSpec((C_OUT, D_MODEL), lambda b, i: (0, 0)),
            pl.BlockSpec((1, C_OUT), lambda b, i: (0, 0)),
        ],
        out_specs=pl.BlockSpec((1, RB, C_OUT), lambda b, i: (b, i, 0)),
        out_shape=jax.ShapeDtypeStruct((BATCH, L_TOT, C_OUT), F32),
    )(enc, gf, bf, wp, bp)


CGN = 128


def _sc_sort_gather(undo, qkvr):
    mesh = plsc.VectorSubcoreMesh(core_axis_name="c", subcore_axis_name="s")

    @functools.partial(
        pl.kernel,
        mesh=mesh,
        out_type=[
            jax.ShapeDtypeStruct((BH, NS), jnp.int32),
            jax.ShapeDtypeStruct((BH, NS, 2 * DH), F32),
        ],
        scratch_types=[
            pltpu.VMEM((NS,), jnp.int32),
            pltpu.VMEM((NS,), jnp.int32),
            pltpu.VMEM((CGN, 2 * DH), F32),
            pltpu.SemaphoreType.DMA,
        ],
        compiler_params=pltpu.CompilerParams(needs_layout_passes=False),
    )
    def k(undo_hbm, qkv_hbm, st_hbm, sqkv_hbm, undo_v, st_v, buf, sem):
        w = lax.axis_index("s") * 2 + lax.axis_index("c")
        pltpu.sync_copy(undo_hbm.at[w], undo_v)

        def scat(i, _):
            idx = undo_v[pl.ds(i * 16, 16)]
            val = (lax.iota(jnp.int32, 16) + i * 16) & (L_TOT - 1)
            plsc.store_scatter(st_v, [idx], val)
            return 0

        lax.fori_loop(0, NS // 16, scat, 0)
        pltpu.sync_copy(st_v, st_hbm.at[w])

        def gath(j, _):
            pltpu.async_copy(
                qkv_hbm.at[w].at[st_v.at[pl.ds(j * CGN, CGN)]], buf, sem).wait()
            pltpu.sync_copy(buf, sqkv_hbm.at[w, pl.ds(j * CGN, CGN)])
            return 0

        lax.fori_loop(0, NS // CGN, gath, 0)

    return k(undo, qkvr)


def _sc_unsort_gather(sop, undo):
    mesh = plsc.VectorSubcoreMesh(core_axis_name="c", subcore_axis_name="s")
    D = 2 * DH

    @functools.partial(
        pl.kernel,
        mesh=mesh,
        out_type=jax.ShapeDtypeStruct((BH, NS, D), F32),
        scratch_types=[
            pltpu.VMEM((NS,), jnp.int32),
            pltpu.VMEM((CGN, D), F32),
            pltpu.SemaphoreType.DMA,
        ],
    )
    def k(sop_hbm, undo_hbm, og_hbm, undo_v, buf, sem):
        w = lax.axis_index("s") * 2 + lax.axis_index("c")
        pltpu.sync_copy(undo_hbm.at[w], undo_v)

        def gath(j, _):
            pltpu.async_copy(
                sop_hbm.at[w].at[undo_v.at[pl.ds(j * CGN, CGN)]], buf, sem).wait()
            pltpu.sync_copy(buf, og_hbm.at[w, pl.ds(j * CGN, CGN)])
            return 0

        lax.fori_loop(0, NS // CGN, gath, 0)

    return k(sop, undo)


def _layer(enc, p, rm):
    wqk_h = p['Wqk'].reshape(N_HEADS, DH, D_MODEL)
    wv_h = p['Wv'].reshape(N_HEADS, DH, D_MODEL)
    qkv, bkt = _qkv_buckets(enc, wqk_h, wv_h, rm)
    qkvr = qkv.reshape(BH, L_TOT, 2 * DH)
    bktr = bkt.reshape(BH, NS, 1)
    undo = _undo_sort(bktr)[:, :, 0]
    st, sqkv = _sc_sort_gather(undo, qkvr)
    sop = _attention(sqkv, st)
    og = _sc_unsort_gather(sop, undo)
    attn = _combine(og.reshape(BATCH, N_HEADS, N_HASHES, L_TOT, 2 * DH))
    xr = _post_attn(attn, enc, p['Wo'], p['bo'].reshape(1, D_MODEL),
                    p['g1'].reshape(1, D_MODEL), p['bn1'].reshape(1, D_MODEL))
    return _ffn(xr, p['W1'], p['b1'].reshape(1, D_FF), p['W2'],
                p['b2'].reshape(1, D_MODEL), p['g2'].reshape(1, D_MODEL),
                p['bn2'].reshape(1, D_MODEL))


def kernel(x_enc, x_mark_enc, x_dec, x_mark_dec, params):
    x = jnp.concatenate([x_enc, x_dec[:, -PRED_LEN:, :]], axis=1)
    xm = jnp.concatenate([x_mark_enc, x_mark_dec[:, -PRED_LEN:, :]], axis=1)
    B, L, _ = x.shape
    W = params['conv_token']
    xin = jnp.concatenate([jnp.roll(x, 1, axis=1), x, jnp.roll(x, -1, axis=1), xm],
                          axis=-1)
    w_in = jnp.concatenate([W[:, :, 0].T, W[:, :, 1].T, W[:, :, 2].T,
                            params['W_timef'].T], axis=0)
    pe = _pos_embed(L, D_MODEL)
    pe2 = jnp.broadcast_to(pe[None], (B, L, D_MODEL)).reshape(B * L, D_MODEL)
    enc = _embed(xin.reshape(B * L, -1), w_in, pe2).reshape(B, L, D_MODEL)

    for li, p in enumerate(params['layers']):
        k = jax.random.fold_in(jax.random.key(42), li)
        rot = jax.random.normal(k, (1, DH, N_HASHES, NBK // 2), F32)
        rm = rot[0].reshape(DH, N_HASHES * (NBK // 2))
        enc = _layer(enc, p, rm)

    dec = _final(enc, params['gf'].reshape(1, D_MODEL),
                 params['bf'].reshape(1, D_MODEL), params['Wp'],
                 params['bp'].reshape(1, C_OUT))
    return dec[:, -PRED_LEN:, :]


# speedup vs baseline: 2.1000x; 1.0845x over previous
import functools
import numpy as np

import jax
import jax.numpy as jnp
from jax import lax
from jax.experimental import pallas as pl
from jax.experimental.pallas import tpu as pltpu
from jax.experimental.pallas import tpu_sc as plsc

D_MODEL = 1024
N_HEADS = 16
D_FF = 4096
ENC_IN = 21
C_OUT = 21
SEQ_LEN = 1536
PRED_LEN = 512
BUCKET = 64
N_HASHES = 4
L_TOT = SEQ_LEN + PRED_LEN
DH = D_MODEL // N_HEADS
NBK = L_TOT // BUCKET
NB = N_HASHES * NBK
NS = N_HASHES * L_TOT
NC = NS // BUCKET
BATCH = 2
BH = BATCH * N_HEADS
RB = 512
F32 = jnp.float32
BF16 = jnp.bfloat16


def _bdot(a, b, dims):
    return lax.dot_general(a.astype(BF16), b.astype(BF16), (dims, ((), ())),
                           preferred_element_type=F32)


def _pos_embed(L, d):
    position = jnp.arange(L, dtype=F32)[:, None]
    div = jnp.exp(jnp.arange(0, d, 2, dtype=F32) * (-np.log(10000.0) / d))
    pe = jnp.zeros((L, d), F32)
    pe = pe.at[:, 0::2].set(jnp.sin(position * div))
    pe = pe.at[:, 1::2].set(jnp.cos(position * div))
    return pe


def _embed_kern(xin_ref, w_ref, pe_ref, o_ref):
    o_ref[...] = _bdot(xin_ref[...], w_ref[...], ((1,), (0,))) + pe_ref[...]


def _embed(xin, w_in, pe):
    n = xin.shape[0] // RB
    c3 = xin.shape[1]
    return pl.pallas_call(
        _embed_kern,
        grid=(n,),
        in_specs=[
            pl.BlockSpec((RB, c3), lambda i: (i, 0)),
            pl.BlockSpec((c3, D_MODEL), lambda i: (0, 0)),
            pl.BlockSpec((RB, D_MODEL), lambda i: (i, 0)),
        ],
        out_specs=pl.BlockSpec((RB, D_MODEL), lambda i: (i, 0)),
        out_shape=jax.ShapeDtypeStruct((xin.shape[0], D_MODEL), F32),
    )(xin, w_in, pe)


def _qkv_kern(x_ref, wqk_ref, wv_ref, rm_ref, qkv_ref, bkt_ref):
    x = x_ref[0]
    qk = _bdot(x, wqk_ref[0], ((1,), (1,)))
    v = _bdot(x, wv_ref[0], ((1,), (1,)))
    qkv_ref[0, 0, :, 0:DH] = qk
    qkv_ref[0, 0, :, DH:2 * DH] = v
    rot = _bdot(qk, rm_ref[...], ((1,), (0,)))
    for g in range(N_HASHES):
        r = rot[:, g * (NBK // 2):(g + 1) * (NBK // 2)]
        c = jnp.concatenate([r, -r], axis=1)
        mx = jnp.max(c, axis=1, keepdims=True)
        io = lax.broadcasted_iota(jnp.int32, (L_TOT, NBK), 1)
        idx = jnp.min(jnp.where(c == mx, io, NB), axis=1, keepdims=True)
        bkt_ref[0, 0, g, :, 0:1] = idx + g * NBK


def _qkv_buckets(enc, wqk_h, wv_h, rm):
    return pl.pallas_call(
        _qkv_kern,
        grid=(BATCH, N_HEADS),
        in_specs=[
            pl.BlockSpec((1, L_TOT, D_MODEL), lambda b, h: (b, 0, 0)),
            pl.BlockSpec((1, DH, D_MODEL), lambda b, h: (h, 0, 0)),
            pl.BlockSpec((1, DH, D_MODEL), lambda b, h: (h, 0, 0)),
            pl.BlockSpec((DH, DH), lambda b, h: (0, 0)),
        ],
        out_specs=[
            pl.BlockSpec((1, 1, L_TOT, 2 * DH), lambda b, h: (b, h, 0, 0)),
            pl.BlockSpec((1, 1, N_HASHES, L_TOT, 1), lambda b, h: (b, h, 0, 0, 0)),
        ],
        out_shape=[
            jax.ShapeDtypeStruct((BATCH, N_HEADS, L_TOT, 2 * DH), F32),
            jax.ShapeDtypeStruct((BATCH, N_HEADS, N_HASHES, L_TOT, 1), jnp.int32),
        ],
    )(enc, wqk_h, wv_h, rm)


CH = 1024


def _undo_kern(bkt_ref, lmat_ref, undo_ref):
    io_l = lax.broadcasted_iota(jnp.int32, (CH, NB), 1)

    def count_body(c, counts):
        bc = bkt_ref[0, pl.ds(c * CH, CH), :]
        oh = (bc == io_l).astype(F32)
        return counts + jnp.sum(oh, axis=0, keepdims=True)

    counts = lax.fori_loop(0, NS // CH, count_body, jnp.zeros((1, NB), F32))
    s = counts
    for sh in (1, 2, 4, 8, 16, 32, 64):
        s = s + jnp.concatenate([jnp.zeros((1, sh), F32), s[:, :NB - sh]], axis=1)
    cum_excl = s - counts

    def rank_body(c, run):
        bc = bkt_ref[0, pl.ds(c * CH, CH), :]
        ohb = bc == io_l
        oh = ohb.astype(F32)
        rank = lax.dot_general(lmat_ref[...], oh.astype(BF16),
                               (((1,), (0,)), ((), ())),
                               preferred_element_type=F32)
        base = cum_excl + run
        pos = jnp.sum(jnp.where(ohb, rank + base, 0.0), axis=1, keepdims=True)
        undo_ref[0, pl.ds(c * CH, CH), :] = pos.astype(jnp.int32)
        return run + jnp.sum(oh, axis=0, keepdims=True)

    lax.fori_loop(0, NS // CH, rank_body, jnp.zeros((1, NB), F32))


def _undo_sort(bkt):
    lmat = jnp.asarray(np.tril(np.ones((CH, CH), np.float32), -1), BF16)
    return pl.pallas_call(
        _undo_kern,
        grid=(BH,),
        in_specs=[pl.BlockSpec((1, NS, 1), lambda i: (i, 0, 0)),
                  pl.BlockSpec((CH, CH), lambda i: (0, 0))],
        out_specs=pl.BlockSpec((1, NS, 1), lambda i: (i, 0, 0)),
        out_shape=jax.ShapeDtypeStruct((BH, NS, 1), jnp.int32),
    )(bkt, lmat)


SCH = 256
SKV = SCH + BUCKET
NSC = NS // SCH


def _attn_kern(sqkv_ref, stc_ref, str_ref, sop_ref):
    qi = lax.broadcasted_iota(jnp.int32, (SCH, SKV), 0) // BUCKET
    kj = lax.broadcasted_iota(jnp.int32, (SCH, SKV), 1) // BUCKET
    d = qi + 1 - kj
    outside = jnp.logical_or(d < 0, d > 1)

    def do_super(s, kv, kt):
        q = sqkv_ref[0, pl.ds(s * SCH, SCH), 0:DH]
        k = kv[:, 0:DH]
        vv = kv[:, DH:2 * DH]
        nrm = jnp.sqrt(jnp.sum(k * k, axis=1, keepdims=True))
        kn = k / jnp.maximum(nrm, 1e-12)
        dots = _bdot(q, kn, ((1,), (1,))) * (DH ** -0.5)
        qt = stc_ref[0, pl.ds(s * SCH, SCH), :]
        dots = jnp.where(qt == kt, -5e4, dots)
        dots = jnp.where(outside, -1e9, dots)
        mx = jnp.max(dots, axis=1, keepdims=True)
        ex = jnp.exp(dots - mx)
        lse = mx + jnp.log(jnp.sum(ex, axis=1, keepdims=True))
        probs = jnp.exp(dots - lse)
        bo = _bdot(probs, vv, ((1,), (0,)))
        sop_ref[0, pl.ds(s * SCH, SCH), 0:DH] = bo
        sop_ref[0, pl.ds(s * SCH, SCH), DH:DH + 16] = jnp.broadcast_to(
            lse, (SCH, 16))

    nps = SCH // BUCKET
    def kt_row(p):
        return str_ref[0, pl.ds(p, 1), :]

    kv0 = jnp.concatenate([sqkv_ref[0, pl.ds(NS - BUCKET, BUCKET), :],
                           sqkv_ref[0, pl.ds(0, SCH), :]], axis=0)
    kt0 = jnp.concatenate([kt_row(NC - 1)] + [kt_row(p) for p in range(nps)],
                          axis=1)
    do_super(0, kv0, kt0)

    def body(s, _):
        kv = sqkv_ref[0, pl.ds(s * SCH - BUCKET, SKV), :]
        kt = jnp.concatenate([kt_row(s * nps - 1 + p) for p in range(nps + 1)],
                             axis=1)
        do_super(s, kv, kt)
        return 0

    lax.fori_loop(1, NSC, body, 0)


def _attention(sqkv, st):
    stc = st.reshape(BH, NS, 1)
    strow = st.reshape(BH, NC, BUCKET)
    return pl.pallas_call(
        _attn_kern,
        grid=(BH,),
        in_specs=[
            pl.BlockSpec((1, NS, 2 * DH), lambda i: (i, 0, 0)),
            pl.BlockSpec((1, NS, 1), lambda i: (i, 0, 0)),
            pl.BlockSpec((1, NC, BUCKET), lambda i: (i, 0, 0)),
        ],
        out_specs=pl.BlockSpec((1, NS, 2 * DH), lambda i: (i, 0, 0)),
        out_shape=jax.ShapeDtypeStruct((BH, NS, 2 * DH), F32),
    )(sqkv, stc, strow)


def _combine_kern(og_ref, out_ref):
    for hh in range(2):
        lgs = [og_ref[0, hh, g, :, DH:DH + 1] for g in range(N_HASHES)]
        mx = lgs[0]
        for g in range(1, N_HASHES):
            mx = jnp.maximum(mx, lgs[g])
        ssum = sum(jnp.exp(lg - mx) for lg in lgs)
        lse = mx + jnp.log(ssum)
        acc = jnp.zeros((L_TOT, DH), F32)
        for g in range(N_HASHES):
            acc = acc + jnp.exp(lgs[g] - lse) * og_ref[0, hh, g, :, 0:DH]
        out_ref[0, :, hh * DH:(hh + 1) * DH] = acc


def _combine(og):
    return pl.pallas_call(
        _combine_kern,
        grid=(BATCH, N_HEADS // 2),
        in_specs=[pl.BlockSpec((1, 2, N_HASHES, L_TOT, 2 * DH),
                               lambda b, h: (b, h, 0, 0, 0))],
        out_specs=pl.BlockSpec((1, L_TOT, 2 * DH), lambda b, h: (b, 0, h)),
        out_shape=jax.ShapeDtypeStruct((BATCH, L_TOT, D_MODEL), F32),
    )(og)


def _ln(x, g, b):
    m = jnp.mean(x, axis=1, keepdims=True)
    v = jnp.mean((x - m) ** 2, axis=1, keepdims=True)
    return (x - m) / jnp.sqrt(v + 1e-5) * g + b


def _post_kern(attn_ref, enc_ref, wo_ref, bo_ref, g1_ref, bn1_ref, xr_ref):
    acc = _bdot(attn_ref[0], wo_ref[...], ((1,), (1,)))
    acc = acc + bo_ref[...] + enc_ref[0]
    xr_ref[0] = _ln(acc, g1_ref[...], bn1_ref[...])


def _post_attn(attn, enc, wo, bo, g1, bn1):
    n = L_TOT // RB
    return pl.pallas_call(
        _post_kern,
        grid=(BATCH, n),
        in_specs=[
            pl.BlockSpec((1, RB, D_MODEL), lambda b, i: (b, i, 0)),
            pl.BlockSpec((1, RB, D_MODEL), lambda b, i: (b, i, 0)),
            pl.BlockSpec((D_MODEL, D_MODEL), lambda b, i: (0, 0)),
            pl.BlockSpec((1, D_MODEL), lambda b, i: (0, 0)),
            pl.BlockSpec((1, D_MODEL), lambda b, i: (0, 0)),
            pl.BlockSpec((1, D_MODEL), lambda b, i: (0, 0)),
        ],
        out_specs=pl.BlockSpec((1, RB, D_MODEL), lambda b, i: (b, i, 0)),
        out_shape=jax.ShapeDtypeStruct((BATCH, L_TOT, D_MODEL), F32),
    )(attn, enc, wo, bo, g1, bn1)


FB = D_FF // 1024


def _ffn_kern(xr_ref, w1_ref, b1_ref, w2_ref, b2_ref, g2_ref, bn2_ref, out_ref):
    j = pl.program_id(2)
    y = _bdot(xr_ref[0], w1_ref[...], ((1,), (1,))) + b1_ref[...]
    y = 0.5 * y * (lax.erf(y * (2 ** -0.5)) + 1.0)
    part = _bdot(y, w2_ref[...], ((1,), (1,)))

    @pl.when(j == 0)
    def _():
        out_ref[0] = part

    @pl.when(j > 0)
    def _():
        out_ref[0] += part

    @pl.when(j == FB - 1)
    def _():
        acc = out_ref[0] + b2_ref[...] + xr_ref[0]
        out_ref[0] = _ln(acc, g2_ref[...], bn2_ref[...])


def _ffn(xr, w1, b1, w2, b2, g2, bn2):
    n = L_TOT // RB
    return pl.pallas_call(
        _ffn_kern,
        grid=(BATCH, n, FB),
        in_specs=[
            pl.BlockSpec((1, RB, D_MODEL), lambda b, i, j: (b, i, 0)),
            pl.BlockSpec((1024, D_MODEL), lambda b, i, j: (j, 0)),
            pl.BlockSpec((1, 1024), lambda b, i, j: (0, j)),
            pl.BlockSpec((D_MODEL, 1024), lambda b, i, j: (0, j)),
            pl.BlockSpec((1, D_MODEL), lambda b, i, j: (0, 0)),
            pl.BlockSpec((1, D_MODEL), lambda b, i, j: (0, 0)),
            pl.BlockSpec((1, D_MODEL), lambda b, i, j: (0, 0)),
        ],
        out_specs=pl.BlockSpec((1, RB, D_MODEL), lambda b, i, j: (b, i, 0)),
        out_shape=jax.ShapeDtypeStruct((BATCH, L_TOT, D_MODEL), F32),
    )(xr, w1, b1, w2, b2, g2, bn2)


def _final_kern(enc_ref, gf_ref, bf_ref, wp_ref, bp_ref, out_ref):
    x = _ln(enc_ref[0], gf_ref[...], bf_ref[...])
    out_ref[0] = _bdot(x, wp_ref[...], ((1,), (1,))) + bp_ref[...]


def _final(enc, gf, bf, wp, bp):
    n = L_TOT // RB
    return pl.pallas_call(
        _final_kern,
        grid=(BATCH, n),
        in_specs=[
            pl.BlockSpec((1, RB, D_MODEL), lambda b, i: (b, i, 0)),
            pl.BlockSpec((1, D_MODEL), lambda b, i: (0, 0)),
            pl.BlockSpec((1, D_MODEL), lambda b, i: (0, 0)),
            pl.BlockSpec((C_OUT, D_MODEL), lambda b, i: (0, 0)),
            pl.BlockSpec((1, C_OUT), lambda b, i: (0, 0)),
        ],
        out_specs=pl.BlockSpec((1, RB, C_OUT), lambda b, i: (b, i, 0)),
        out_shape=jax.ShapeDtypeStruct((BATCH, L_TOT, C_OUT), F32),
    )(enc, gf, bf, wp, bp)


CGN = 128


def _sc_sort_gather(undo, qkvr):
    mesh = plsc.VectorSubcoreMesh(core_axis_name="c", subcore_axis_name="s")

    @functools.partial(
        pl.kernel,
        mesh=mesh,
        out_type=[
            jax.ShapeDtypeStruct((BH, NS), jnp.int32),
            jax.ShapeDtypeStruct((BH, NS, 2 * DH), F32),
        ],
        scratch_types=[
            pltpu.VMEM((NS,), jnp.int32),
            pltpu.VMEM((NS,), jnp.int32),
            pltpu.VMEM((CGN, 2 * DH), F32),
            pltpu.SemaphoreType.DMA,
        ],
        compiler_params=pltpu.CompilerParams(needs_layout_passes=False),
    )
    def k(undo_hbm, qkv_hbm, st_hbm, sqkv_hbm, undo_v, st_v, buf, sem):
        w = lax.axis_index("s") * 2 + lax.axis_index("c")
        pltpu.sync_copy(undo_hbm.at[w], undo_v)

        def scat(i, _):
            idx = undo_v[pl.ds(i * 16, 16)]
            val = (lax.iota(jnp.int32, 16) + i * 16) & (L_TOT - 1)
            plsc.store_scatter(st_v, [idx], val)
            return 0

        lax.fori_loop(0, NS // 16, scat, 0)
        pltpu.sync_copy(st_v, st_hbm.at[w])

        def gath(j, _):
            pltpu.async_copy(
                qkv_hbm.at[w].at[st_v.at[pl.ds(j * CGN, CGN)]], buf, sem).wait()
            pltpu.sync_copy(buf, sqkv_hbm.at[w, pl.ds(j * CGN, CGN)])
            return 0

        lax.fori_loop(0, NS // CGN, gath, 0)

    return k(undo, qkvr)


def _sc_unsort_gather(sop, undo):
    mesh = plsc.VectorSubcoreMesh(core_axis_name="c", subcore_axis_name="s")
    D = 2 * DH

    @functools.partial(
        pl.kernel,
        mesh=mesh,
        out_type=jax.ShapeDtypeStruct((BH, NS, D), F32),
        scratch_types=[
            pltpu.VMEM((NS,), jnp.int32),
            pltpu.VMEM((CGN, D), F32),
            pltpu.SemaphoreType.DMA,
        ],
    )
    def k(sop_hbm, undo_hbm, og_hbm, undo_v, buf, sem):
        w = lax.axis_index("s") * 2 + lax.axis_index("c")
        pltpu.sync_copy(undo_hbm.at[w], undo_v)

        def gath(j, _):
            pltpu.async_copy(
                sop_hbm.at[w].at[undo_v.at[pl.ds(j * CGN, CGN)]], buf, sem).wait()
            pltpu.sync_copy(buf, og_hbm.at[w, pl.ds(j * CGN, CGN)])
            return 0

        lax.fori_loop(0, NS // CGN, gath, 0)

    return k(sop, undo)


def _layer(enc, p, rm):
    wqk_h = p['Wqk'].reshape(N_HEADS, DH, D_MODEL)
    wv_h = p['Wv'].reshape(N_HEADS, DH, D_MODEL)
    qkv, bkt = _qkv_buckets(enc, wqk_h, wv_h, rm)
    qkvr = qkv.reshape(BH, L_TOT, 2 * DH)
    bktr = bkt.reshape(BH, NS, 1)
    undo = _undo_sort(bktr)[:, :, 0]
    st, sqkv = _sc_sort_gather(undo, qkvr)
    sop = _attention(sqkv, st)
    og = _sc_unsort_gather(sop, undo)
    attn = _combine(og.reshape(BATCH, N_HEADS, N_HASHES, L_TOT, 2 * DH))
    xr = _post_attn(attn, enc, p['Wo'], p['bo'].reshape(1, D_MODEL),
                    p['g1'].reshape(1, D_MODEL), p['bn1'].reshape(1, D_MODEL))
    return _ffn(xr, p['W1'], p['b1'].reshape(1, D_FF), p['W2'],
                p['b2'].reshape(1, D_MODEL), p['g2'].reshape(1, D_MODEL),
                p['bn2'].reshape(1, D_MODEL))


def kernel(x_enc, x_mark_enc, x_dec, x_mark_dec, params):
    x = jnp.concatenate([x_enc, x_dec[:, -PRED_LEN:, :]], axis=1)
    xm = jnp.concatenate([x_mark_enc, x_mark_dec[:, -PRED_LEN:, :]], axis=1)
    B, L, _ = x.shape
    W = params['conv_token']
    xin = jnp.concatenate([jnp.roll(x, 1, axis=1), x, jnp.roll(x, -1, axis=1), xm],
                          axis=-1)
    w_in = jnp.concatenate([W[:, :, 0].T, W[:, :, 1].T, W[:, :, 2].T,
                            params['W_timef'].T], axis=0)
    pe = _pos_embed(L, D_MODEL)
    pe2 = jnp.broadcast_to(pe[None], (B, L, D_MODEL)).reshape(B * L, D_MODEL)
    enc = _embed(xin.reshape(B * L, -1), w_in, pe2).reshape(B, L, D_MODEL)

    for li, p in enumerate(params['layers']):
        k = jax.random.fold_in(jax.random.key(42), li)
        rot = jax.random.normal(k, (1, DH, N_HASHES, NBK // 2), F32)
        rm = rot[0].reshape(DH, N_HASHES * (NBK // 2))
        enc = _layer(enc, p, rm)

    dec = _final(enc, params['gf'].reshape(1, D_MODEL),
                 params['bf'].reshape(1, D_MODEL), params['Wp'],
                 params['bp'].reshape(1, C_OUT))
    return dec[:, -PRED_LEN:, :]


# speedup vs baseline: 2.1891x; 1.0424x over previous
import functools
import numpy as np

import jax
import jax.numpy as jnp
from jax import lax
from jax.experimental import pallas as pl
from jax.experimental.pallas import tpu as pltpu
from jax.experimental.pallas import tpu_sc as plsc

D_MODEL = 1024
N_HEADS = 16
D_FF = 4096
ENC_IN = 21
C_OUT = 21
SEQ_LEN = 1536
PRED_LEN = 512
BUCKET = 64
N_HASHES = 4
L_TOT = SEQ_LEN + PRED_LEN
DH = D_MODEL // N_HEADS
NBK = L_TOT // BUCKET
NB = N_HASHES * NBK
NS = N_HASHES * L_TOT
NC = NS // BUCKET
BATCH = 2
BH = BATCH * N_HEADS
RB = 512
F32 = jnp.float32
BF16 = jnp.bfloat16


def _bdot(a, b, dims):
    return lax.dot_general(a.astype(BF16), b.astype(BF16), (dims, ((), ())),
                           preferred_element_type=F32)


def _pos_embed(L, d):
    position = jnp.arange(L, dtype=F32)[:, None]
    div = jnp.exp(jnp.arange(0, d, 2, dtype=F32) * (-np.log(10000.0) / d))
    pe = jnp.zeros((L, d), F32)
    pe = pe.at[:, 0::2].set(jnp.sin(position * div))
    pe = pe.at[:, 1::2].set(jnp.cos(position * div))
    return pe


def _embed_kern(xin_ref, w_ref, pe_ref, o_ref):
    o_ref[...] = _bdot(xin_ref[...], w_ref[...], ((1,), (0,))) + pe_ref[...]


def _embed(xin, w_in, pe):
    n = xin.shape[0] // RB
    c3 = xin.shape[1]
    return pl.pallas_call(
        _embed_kern,
        grid=(n,),
        in_specs=[
            pl.BlockSpec((RB, c3), lambda i: (i, 0)),
            pl.BlockSpec((c3, D_MODEL), lambda i: (0, 0)),
            pl.BlockSpec((RB, D_MODEL), lambda i: (i, 0)),
        ],
        out_specs=pl.BlockSpec((RB, D_MODEL), lambda i: (i, 0)),
        out_shape=jax.ShapeDtypeStruct((xin.shape[0], D_MODEL), F32),
    )(xin, w_in, pe)


def _qkv_kern(x_ref, wqk_ref, wv_ref, rm_ref, qkv_ref, bkt_ref):
    x = x_ref[0]
    qk = _bdot(x, wqk_ref[0], ((1,), (1,)))
    v = _bdot(x, wv_ref[0], ((1,), (1,)))
    qkv_ref[0, 0, :, 0:DH] = qk
    qkv_ref[0, 0, :, DH:2 * DH] = v
    rot = _bdot(qk, rm_ref[...], ((1,), (0,)))
    for g in range(N_HASHES):
        r = rot[:, g * (NBK // 2):(g + 1) * (NBK // 2)]
        c = jnp.concatenate([r, -r], axis=1)
        mx = jnp.max(c, axis=1, keepdims=True)
        io = lax.broadcasted_iota(jnp.int32, (L_TOT, NBK), 1)
        idx = jnp.min(jnp.where(c == mx, io, NB), axis=1, keepdims=True)
        bkt_ref[0, 0, g, :, 0:1] = idx + g * NBK


def _qkv_buckets(enc, wqk_h, wv_h, rm):
    return pl.pallas_call(
        _qkv_kern,
        grid=(BATCH, N_HEADS),
        in_specs=[
            pl.BlockSpec((1, L_TOT, D_MODEL), lambda b, h: (b, 0, 0)),
            pl.BlockSpec((1, DH, D_MODEL), lambda b, h: (h, 0, 0)),
            pl.BlockSpec((1, DH, D_MODEL), lambda b, h: (h, 0, 0)),
            pl.BlockSpec((DH, DH), lambda b, h: (0, 0)),
        ],
        out_specs=[
            pl.BlockSpec((1, 1, L_TOT, 2 * DH), lambda b, h: (b, h, 0, 0)),
            pl.BlockSpec((1, 1, N_HASHES, L_TOT, 1), lambda b, h: (b, h, 0, 0, 0)),
        ],
        out_shape=[
            jax.ShapeDtypeStruct((BATCH, N_HEADS, L_TOT, 2 * DH), F32),
            jax.ShapeDtypeStruct((BATCH, N_HEADS, N_HASHES, L_TOT, 1), jnp.int32),
        ],
    )(enc, wqk_h, wv_h, rm)


CH = 1024


def _undo_kern(bkt_ref, lmat_ref, undo_ref):
    io_l = lax.broadcasted_iota(jnp.int32, (CH, NB), 1)

    def count_body(c, counts):
        bc = bkt_ref[0, pl.ds(c * CH, CH), :]
        oh = (bc == io_l).astype(F32)
        return counts + jnp.sum(oh, axis=0, keepdims=True)

    counts = lax.fori_loop(0, NS // CH, count_body, jnp.zeros((1, NB), F32))
    s = counts
    for sh in (1, 2, 4, 8, 16, 32, 64):
        s = s + jnp.concatenate([jnp.zeros((1, sh), F32), s[:, :NB - sh]], axis=1)
    cum_excl = s - counts

    def rank_body(c, run):
        bc = bkt_ref[0, pl.ds(c * CH, CH), :]
        ohb = bc == io_l
        oh = ohb.astype(F32)
        rank = lax.dot_general(lmat_ref[...], oh.astype(BF16),
                               (((1,), (0,)), ((), ())),
                               preferred_element_type=F32)
        base = cum_excl + run
        pos = jnp.sum(jnp.where(ohb, rank + base, 0.0), axis=1, keepdims=True)
        undo_ref[0, pl.ds(c * CH, CH), :] = pos.astype(jnp.int32)
        return run + jnp.sum(oh, axis=0, keepdims=True)

    lax.fori_loop(0, NS // CH, rank_body, jnp.zeros((1, NB), F32))


def _undo_sort(bkt):
    lmat = jnp.asarray(np.tril(np.ones((CH, CH), np.float32), -1), BF16)
    return pl.pallas_call(
        _undo_kern,
        grid=(BH,),
        in_specs=[pl.BlockSpec((1, NS, 1), lambda i: (i, 0, 0)),
                  pl.BlockSpec((CH, CH), lambda i: (0, 0))],
        out_specs=pl.BlockSpec((1, NS, 1), lambda i: (i, 0, 0)),
        out_shape=jax.ShapeDtypeStruct((BH, NS, 1), jnp.int32),
    )(bkt, lmat)


SCH = 256
SKV = SCH + BUCKET
NSC = NS // SCH


def _attn_kern(sqkv_ref, stc_ref, str_ref, sop_ref):
    qi = lax.broadcasted_iota(jnp.int32, (SCH, SKV), 0) // BUCKET
    kj = lax.broadcasted_iota(jnp.int32, (SCH, SKV), 1) // BUCKET
    d = qi + 1 - kj
    outside = jnp.logical_or(d < 0, d > 1)

    def do_super(s, kv, kt):
        q = sqkv_ref[0, pl.ds(s * SCH, SCH), 0:DH]
        k = kv[:, 0:DH]
        vv = kv[:, DH:2 * DH]
        nrm = jnp.sqrt(jnp.sum(k * k, axis=1, keepdims=True))
        kn = k / jnp.maximum(nrm, 1e-12)
        dots = _bdot(q, kn, ((1,), (1,))) * (DH ** -0.5)
        qt = stc_ref[0, pl.ds(s * SCH, SCH), :]
        dots = jnp.where(qt == kt, -5e4, dots)
        dots = jnp.where(outside, -1e9, dots)
        mx = jnp.max(dots, axis=1, keepdims=True)
        ex = jnp.exp(dots - mx)
        lse = mx + jnp.log(jnp.sum(ex, axis=1, keepdims=True))
        probs = jnp.exp(dots - lse)
        bo = _bdot(probs, vv, ((1,), (0,)))
        sop_ref[0, pl.ds(s * SCH, SCH), 0:DH] = bo
        sop_ref[0, pl.ds(s * SCH, SCH), DH:DH + 16] = jnp.broadcast_to(
            lse, (SCH, 16))

    nps = SCH // BUCKET
    def kt_row(p):
        return str_ref[0, pl.ds(p, 1), :]

    kv0 = jnp.concatenate([sqkv_ref[0, pl.ds(NS - BUCKET, BUCKET), :],
                           sqkv_ref[0, pl.ds(0, SCH), :]], axis=0)
    kt0 = jnp.concatenate([kt_row(NC - 1)] + [kt_row(p) for p in range(nps)],
                          axis=1)
    do_super(0, kv0, kt0)

    def body(s, _):
        kv = sqkv_ref[0, pl.ds(s * SCH - BUCKET, SKV), :]
        kt = jnp.concatenate([kt_row(s * nps - 1 + p) for p in range(nps + 1)],
                             axis=1)
        do_super(s, kv, kt)
        return 0

    lax.fori_loop(1, NSC, body, 0)


def _attention(sqkv, st):
    stc = st.reshape(BH, NS, 1)
    strow = st.reshape(BH, NC, BUCKET)
    return pl.pallas_call(
        _attn_kern,
        grid=(BH,),
        in_specs=[
            pl.BlockSpec((1, NS, 2 * DH), lambda i: (i, 0, 0)),
            pl.BlockSpec((1, NS, 1), lambda i: (i, 0, 0)),
            pl.BlockSpec((1, NC, BUCKET), lambda i: (i, 0, 0)),
        ],
        out_specs=pl.BlockSpec((1, NS, 2 * DH), lambda i: (i, 0, 0)),
        out_shape=jax.ShapeDtypeStruct((BH, NS, 2 * DH), F32),
    )(sqkv, stc, strow)


def _combine_kern(og_ref, out_ref):
    for hh in range(2):
        lgs = [og_ref[0, hh, g, :, DH:DH + 1] for g in range(N_HASHES)]
        mx = lgs[0]
        for g in range(1, N_HASHES):
            mx = jnp.maximum(mx, lgs[g])
        ssum = sum(jnp.exp(lg - mx) for lg in lgs)
        lse = mx + jnp.log(ssum)
        acc = jnp.zeros((L_TOT, DH), F32)
        for g in range(N_HASHES):
            acc = acc + jnp.exp(lgs[g] - lse) * og_ref[0, hh, g, :, 0:DH]
        out_ref[0, :, hh * DH:(hh + 1) * DH] = acc


def _combine(og):
    return pl.pallas_call(
        _combine_kern,
        grid=(BATCH, N_HEADS // 2),
        in_specs=[pl.BlockSpec((1, 2, N_HASHES, L_TOT, 2 * DH),
                               lambda b, h: (b, h, 0, 0, 0))],
        out_specs=pl.BlockSpec((1, L_TOT, 2 * DH), lambda b, h: (b, 0, h)),
        out_shape=jax.ShapeDtypeStruct((BATCH, L_TOT, D_MODEL), F32),
    )(og)


def _ln(x, g, b):
    m = jnp.mean(x, axis=1, keepdims=True)
    v = jnp.mean((x - m) ** 2, axis=1, keepdims=True)
    return (x - m) / jnp.sqrt(v + 1e-5) * g + b


def _post_kern(attn_ref, enc_ref, wo_ref, bo_ref, g1_ref, bn1_ref, xr_ref):
    acc = _bdot(attn_ref[0], wo_ref[...], ((1,), (1,)))
    acc = acc + bo_ref[...] + enc_ref[0]
    xr_ref[0] = _ln(acc, g1_ref[...], bn1_ref[...])


def _post_attn(attn, enc, wo, bo, g1, bn1):
    n = L_TOT // RB
    return pl.pallas_call(
        _post_kern,
        grid=(BATCH, n),
        in_specs=[
            pl.BlockSpec((1, RB, D_MODEL), lambda b, i: (b, i, 0)),
            pl.BlockSpec((1, RB, D_MODEL), lambda b, i: (b, i, 0)),
            pl.BlockSpec((D_MODEL, D_MODEL), lambda b, i: (0, 0)),
            pl.BlockSpec((1, D_MODEL), lambda b, i: (0, 0)),
            pl.BlockSpec((1, D_MODEL), lambda b, i: (0, 0)),
            pl.BlockSpec((1, D_MODEL), lambda b, i: (0, 0)),
        ],
        out_specs=pl.BlockSpec((1, RB, D_MODEL), lambda b, i: (b, i, 0)),
        out_shape=jax.ShapeDtypeStruct((BATCH, L_TOT, D_MODEL), F32),
    )(attn, enc, wo, bo, g1, bn1)


FB = D_FF // 1024


def _ffn_kern(xr_ref, w1_ref, b1_ref, w2_ref, b2_ref, g2_ref, bn2_ref, out_ref):
    j = pl.program_id(2)
    y = _bdot(xr_ref[0], w1_ref[...], ((1,), (1,))) + b1_ref[...]
    y = 0.5 * y * (lax.erf(y * (2 ** -0.5)) + 1.0)
    part = _bdot(y, w2_ref[...], ((1,), (1,)))

    @pl.when(j == 0)
    def _():
        out_ref[0] = part

    @pl.when(j > 0)
    def _():
        out_ref[0] += part

    @pl.when(j == FB - 1)
    def _():
        acc = out_ref[0] + b2_ref[...] + xr_ref[0]
        out_ref[0] = _ln(acc, g2_ref[...], bn2_ref[...])


def _ffn(xr, w1, b1, w2, b2, g2, bn2):
    n = L_TOT // RB
    return pl.pallas_call(
        _ffn_kern,
        grid=(BATCH, n, FB),
        in_specs=[
            pl.BlockSpec((1, RB, D_MODEL), lambda b, i, j: (b, i, 0)),
            pl.BlockSpec((1024, D_MODEL), lambda b, i, j: (j, 0)),
            pl.BlockSpec((1, 1024), lambda b, i, j: (0, j)),
            pl.BlockSpec((D_MODEL, 1024), lambda b, i, j: (0, j)),
            pl.BlockSpec((1, D_MODEL), lambda b, i, j: (0, 0)),
            pl.BlockSpec((1, D_MODEL), lambda b, i, j: (0, 0)),
            pl.BlockSpec((1, D_MODEL), lambda b, i, j: (0, 0)),
        ],
        out_specs=pl.BlockSpec((1, RB, D_MODEL), lambda b, i, j: (b, i, 0)),
        out_shape=jax.ShapeDtypeStruct((BATCH, L_TOT, D_MODEL), F32),
    )(xr, w1, b1, w2, b2, g2, bn2)


def _final_kern(enc_ref, gf_ref, bf_ref, wp_ref, bp_ref, out_ref):
    x = _ln(enc_ref[0], gf_ref[...], bf_ref[...])
    out_ref[0] = _bdot(x, wp_ref[...], ((1,), (1,))) + bp_ref[...]


def _final(enc, gf, bf, wp, bp):
    n = L_TOT // RB
    return pl.pallas_call(
        _final_kern,
        grid=(BATCH, n),
        in_specs=[
            pl.BlockSpec((1, RB, D_MODEL), lambda b, i: (b, i, 0)),
            pl.BlockSpec((1, D_MODEL), lambda b, i: (0, 0)),
            pl.BlockSpec((1, D_MODEL), lambda b, i: (0, 0)),
            pl.BlockSpec((C_OUT, D_MODEL), lambda b, i: (0, 0)),
            pl.BlockSpec((1, C_OUT), lambda b, i: (0, 0)),
        ],
        out_specs=pl.BlockSpec((1, RB, C_OUT), lambda b, i: (b, i, 0)),
        out_shape=jax.ShapeDtypeStruct((BATCH, L_TOT, C_OUT), F32),
    )(enc, gf, bf, wp, bp)


CGN = 128


def _db_gather(tbl_row, idx_v, out_hbm, w, bufa, bufb, sema, semb):
    nchunk = NS // CGN

    def start(j, buf, sem):
        pltpu.async_copy(tbl_row.at[idx_v.at[pl.ds(j * CGN, CGN)]], buf, sem)

    def drain(buf, sem):
        pltpu.make_async_copy(out_hbm.at[w, pl.ds(0, CGN)], buf, sem).wait()

    start(0, bufa, sema)

    def body(j, _):
        even = j % 2 == 0

        @pl.when(jnp.logical_and(even, j + 1 < nchunk))
        def _():
            start(j + 1, bufb, semb)

        @pl.when(jnp.logical_and(jnp.logical_not(even), j + 1 < nchunk))
        def _():
            start(j + 1, bufa, sema)

        @pl.when(even)
        def _():
            drain(bufa, sema)
            pltpu.sync_copy(bufa, out_hbm.at[w, pl.ds(j * CGN, CGN)])

        @pl.when(jnp.logical_not(even))
        def _():
            drain(bufb, semb)
            pltpu.sync_copy(bufb, out_hbm.at[w, pl.ds(j * CGN, CGN)])

        return 0

    lax.fori_loop(0, nchunk, body, 0)


def _sc_sort_gather(undo, qkvr):
    mesh = plsc.VectorSubcoreMesh(core_axis_name="c", subcore_axis_name="s")

    @functools.partial(
        pl.kernel,
        mesh=mesh,
        out_type=[
            jax.ShapeDtypeStruct((BH, NS), jnp.int32),
            jax.ShapeDtypeStruct((BH, NS, 2 * DH), F32),
        ],
        scratch_types=[
            pltpu.VMEM((NS,), jnp.int32),
            pltpu.VMEM((NS,), jnp.int32),
            pltpu.VMEM((CGN, 2 * DH), F32),
            pltpu.VMEM((CGN, 2 * DH), F32),
            pltpu.SemaphoreType.DMA,
            pltpu.SemaphoreType.DMA,
        ],
        compiler_params=pltpu.CompilerParams(needs_layout_passes=False),
    )
    def k(undo_hbm, qkv_hbm, st_hbm, sqkv_hbm, undo_v, st_v, bufa, bufb,
          sema, semb):
        w = lax.axis_index("s") * 2 + lax.axis_index("c")
        pltpu.sync_copy(undo_hbm.at[w], undo_v)

        def scat(i, _):
            idx = undo_v[pl.ds(i * 16, 16)]
            val = (lax.iota(jnp.int32, 16) + i * 16) & (L_TOT - 1)
            plsc.store_scatter(st_v, [idx], val)
            return 0

        lax.fori_loop(0, NS // 16, scat, 0)
        pltpu.sync_copy(st_v, st_hbm.at[w])
        _db_gather(qkv_hbm.at[w], st_v, sqkv_hbm, w, bufa, bufb, sema, semb)

    return k(undo, qkvr)


def _sc_unsort_gather(sop, undo):
    mesh = plsc.VectorSubcoreMesh(core_axis_name="c", subcore_axis_name="s")
    D = 2 * DH

    @functools.partial(
        pl.kernel,
        mesh=mesh,
        out_type=jax.ShapeDtypeStruct((BH, NS, D), F32),
        scratch_types=[
            pltpu.VMEM((NS,), jnp.int32),
            pltpu.VMEM((CGN, D), F32),
            pltpu.VMEM((CGN, D), F32),
            pltpu.SemaphoreType.DMA,
            pltpu.SemaphoreType.DMA,
        ],
    )
    def k(sop_hbm, undo_hbm, og_hbm, undo_v, bufa, bufb, sema, semb):
        w = lax.axis_index("s") * 2 + lax.axis_index("c")
        pltpu.sync_copy(undo_hbm.at[w], undo_v)
        _db_gather(sop_hbm.at[w], undo_v, og_hbm, w, bufa, bufb, sema, semb)

    return k(sop, undo)


def _layer(enc, p, rm):
    wqk_h = p['Wqk'].reshape(N_HEADS, DH, D_MODEL)
    wv_h = p['Wv'].reshape(N_HEADS, DH, D_MODEL)
    qkv, bkt = _qkv_buckets(enc, wqk_h, wv_h, rm)
    qkvr = qkv.reshape(BH, L_TOT, 2 * DH)
    bktr = bkt.reshape(BH, NS, 1)
    undo = _undo_sort(bktr)[:, :, 0]
    st, sqkv = _sc_sort_gather(undo, qkvr)
    sop = _attention(sqkv, st)
    og = _sc_unsort_gather(sop, undo)
    attn = _combine(og.reshape(BATCH, N_HEADS, N_HASHES, L_TOT, 2 * DH))
    xr = _post_attn(attn, enc, p['Wo'], p['bo'].reshape(1, D_MODEL),
                    p['g1'].reshape(1, D_MODEL), p['bn1'].reshape(1, D_MODEL))
    return _ffn(xr, p['W1'], p['b1'].reshape(1, D_FF), p['W2'],
                p['b2'].reshape(1, D_MODEL), p['g2'].reshape(1, D_MODEL),
                p['bn2'].reshape(1, D_MODEL))


def kernel(x_enc, x_mark_enc, x_dec, x_mark_dec, params):
    x = jnp.concatenate([x_enc, x_dec[:, -PRED_LEN:, :]], axis=1)
    xm = jnp.concatenate([x_mark_enc, x_mark_dec[:, -PRED_LEN:, :]], axis=1)
    B, L, _ = x.shape
    W = params['conv_token']
    xin = jnp.concatenate([jnp.roll(x, 1, axis=1), x, jnp.roll(x, -1, axis=1), xm],
                          axis=-1)
    w_in = jnp.concatenate([W[:, :, 0].T, W[:, :, 1].T, W[:, :, 2].T,
                            params['W_timef'].T], axis=0)
    pe = _pos_embed(L, D_MODEL)
    pe2 = jnp.broadcast_to(pe[None], (B, L, D_MODEL)).reshape(B * L, D_MODEL)
    enc = _embed(xin.reshape(B * L, -1), w_in, pe2).reshape(B, L, D_MODEL)

    for li, p in enumerate(params['layers']):
        k = jax.random.fold_in(jax.random.key(42), li)
        rot = jax.random.normal(k, (1, DH, N_HASHES, NBK // 2), F32)
        rm = rot[0].reshape(DH, N_HASHES * (NBK // 2))
        enc = _layer(enc, p, rm)

    dec = _final(enc, params['gf'].reshape(1, D_MODEL),
                 params['bf'].reshape(1, D_MODEL), params['Wp'],
                 params['bp'].reshape(1, C_OUT))
    return dec[:, -PRED_LEN:, :]
